# baseline (speedup 1.0000x reference)
import functools

import jax
import jax.numpy as jnp
from jax import lax
from jax.experimental import pallas as pl
from jax.experimental.pallas import tpu as pltpu
from jax.experimental.pallas import tpu_sc as plsc

NC = 2
NS = 16
NW = NC * NS
CHUNK = 128


def _gather_body(nchunk, x0_hbm, x1_hbm, e_hbm, out0_hbm, out1_hbm,
                 idx0_v, idx1_v, rows0_v, rows1_v, sem):
    wid = lax.axis_index("s") * NC + lax.axis_index("c")
    base = wid * nchunk
    pltpu.sync_copy(x0_hbm.at[pl.ds(base, nchunk)], idx0_v)
    pltpu.sync_copy(x1_hbm.at[pl.ds(base, nchunk)], idx1_v)
    copies = []
    for j in range(nchunk):
        copies.append(pltpu.make_async_copy(
            e_hbm.at[idx0_v.at[j]], rows0_v.at[j], sem))
        copies.append(pltpu.make_async_copy(
            e_hbm.at[idx1_v.at[j]], rows1_v.at[j], sem))
    for c in copies:
        c.start()
    for c in copies:
        c.wait()
    pltpu.sync_copy(rows0_v, out0_hbm.at[pl.ds(base, nchunk)])
    pltpu.sync_copy(rows1_v, out1_hbm.at[pl.ds(base, nchunk)])


@functools.partial(jax.jit, static_argnums=(3,))
def _sc_gather(x0, x1, e, bh):
    nchunk = bh // (NW * CHUNK)
    dp = e.shape[1]
    mesh = plsc.VectorSubcoreMesh(core_axis_name="c", subcore_axis_name="s")
    fn = pl.kernel(
        functools.partial(_gather_body, nchunk),
        out_type=(
            jax.ShapeDtypeStruct((NW * nchunk, CHUNK, dp), jnp.float32),
            jax.ShapeDtypeStruct((NW * nchunk, CHUNK, dp), jnp.float32),
        ),
        mesh=mesh,
        scratch_types=[
            pltpu.VMEM((nchunk, CHUNK), jnp.int32),
            pltpu.VMEM((nchunk, CHUNK), jnp.int32),
            pltpu.VMEM((nchunk, CHUNK, dp), jnp.float32),
            pltpu.VMEM((nchunk, CHUNK, dp), jnp.float32),
            pltpu.SemaphoreType.DMA,
        ],
        compiler_params=pltpu.CompilerParams(use_tc_tiling_on_sc=False),
    )
    return fn(x0, x1, e)


def _pack_body(e0t_ref, e1t_ref, i_ref, o_ref):
    d = e0t_ref.shape[0]
    dn = (((0,), (0,)), ((), ()))
    f32 = jnp.float32
    o_ref[:, :d] = lax.dot_general(e0t_ref[...], i_ref[...], dn,
                                   preferred_element_type=f32)
    o_ref[:, d:] = lax.dot_general(e1t_ref[...], i_ref[...], dn,
                                   preferred_element_type=f32)


@jax.jit
def _tc_pack(e0t, e1t):
    d, v = e0t.shape
    bv = 1024
    grid = (pl.cdiv(v, bv),)
    eye = jnp.eye(d, dtype=jnp.float32)
    return pl.pallas_call(
        _pack_body,
        grid=grid,
        in_specs=[
            pl.BlockSpec((d, bv), lambda i: (0, i)),
            pl.BlockSpec((d, bv), lambda i: (0, i)),
            pl.BlockSpec((d, d), lambda i: (0, 0)),
        ],
        out_specs=pl.BlockSpec((bv, 2 * d), lambda i: (i, 0)),
        out_shape=jax.ShapeDtypeStruct((v, 2 * d), jnp.float32),
    )(e0t, e1t, eye)


def _mlp_body(a0_ref, a1_ref, w1a_ref, w1b_ref, b1_ref, w2_ref, b2_ref,
              w3_ref, b3_ref, o_ref):
    f32 = jnp.float32
    d = w1a_ref.shape[0]
    h = (jnp.dot(a0_ref[:, :d], w1a_ref[...], preferred_element_type=f32)
         + jnp.dot(a1_ref[:, d:], w1b_ref[...], preferred_element_type=f32)
         + b1_ref[...])
    h = jnp.maximum(h, 0.0)
    h = jnp.dot(h, w2_ref[...], preferred_element_type=f32) + b2_ref[...]
    h = jnp.maximum(h, 0.0)
    o_ref[...] = jnp.dot(h, w3_ref[...], preferred_element_type=f32) + b3_ref[...]


@jax.jit
def _tc_mlp(e0, e1, w1a, w1b, b1, w2, b2, w3, b3):
    bh, dp = e0.shape
    n_out = w3.shape[1]
    bm = 2048
    grid = (bh // bm,)
    full = lambda shape: pl.BlockSpec(shape, lambda i: (0, 0))
    return pl.pallas_call(
        _mlp_body,
        grid=grid,
        in_specs=[
            pl.BlockSpec((bm, dp), lambda i: (i, 0)),
            pl.BlockSpec((bm, dp), lambda i: (i, 0)),
            full(w1a.shape),
            full(w1b.shape),
            full(b1.shape),
            full(w2.shape),
            full(b2.shape),
            full(w3.shape),
            full(b3.shape),
        ],
        out_specs=pl.BlockSpec((bm, n_out), lambda i: (i, 0)),
        out_shape=jax.ShapeDtypeStruct((bh, n_out), jnp.float32),
    )(e0, e1, w1a, w1b, b1, w2, b2, w3, b3)


def kernel(x, E0, E1, W1, b1, W2, b2, W3, b3):
    b = x.shape[0]
    d = E0.shape[1]
    bh = b // 2
    nchunk = bh // (NW * CHUNK)
    Epk = _tc_pack(jnp.swapaxes(E0, 0, 1), jnp.swapaxes(E1, 0, 1))
    x0 = x[:, 0].reshape(2, NW * nchunk, CHUNK)
    x1 = x[:, 1].reshape(2, NW * nchunk, CHUNK)
    w1a, w1b = W1[:d], W1[d:]
    b1r, b2r, b3r = b1.reshape(1, -1), b2.reshape(1, -1), b3.reshape(1, -1)
    outs = []
    for half in range(2):
        emb0, emb1 = _sc_gather(x0[half], x1[half], Epk, bh)
        outs.append(_tc_mlp(emb0.reshape(bh, 2 * d), emb1.reshape(bh, 2 * d),
                            w1a, w1b, b1r, W2, b2r, W3, b3r))
    return jnp.concatenate(outs, axis=0)


# speedup vs baseline: 1.2289x; 1.2289x over previous
import functools

import jax
import jax.numpy as jnp
from jax import lax
from jax.experimental import pallas as pl
from jax.experimental.pallas import tpu as pltpu
from jax.experimental.pallas import tpu_sc as plsc

NC = 2
NS = 16
NW = NC * NS
CHUNK = 128


def _gather_body(nchunk, x0_hbm, x1_hbm, e_hbm, out0_hbm, out1_hbm,
                 idx0_v, idx1_v, rows0_v, rows1_v, sem):
    wid = lax.axis_index("s") * NC + lax.axis_index("c")
    base = wid * nchunk
    pltpu.sync_copy(x0_hbm.at[pl.ds(base, nchunk)], idx0_v)
    pltpu.sync_copy(x1_hbm.at[pl.ds(base, nchunk)], idx1_v)
    copies = []
    for j in range(nchunk):
        copies.append(pltpu.make_async_copy(
            e_hbm.at[idx0_v.at[j]], rows0_v.at[j], sem))
        copies.append(pltpu.make_async_copy(
            e_hbm.at[idx1_v.at[j]], rows1_v.at[j], sem))
    for c in copies:
        c.start()
    for c in copies:
        c.wait()
    pltpu.sync_copy(rows0_v, out0_hbm.at[pl.ds(base, nchunk)])
    pltpu.sync_copy(rows1_v, out1_hbm.at[pl.ds(base, nchunk)])


@functools.partial(jax.jit, static_argnums=(3,))
def _sc_gather(x0, x1, e, bh):
    nchunk = bh // (NW * CHUNK)
    dp = e.shape[1]
    mesh = plsc.VectorSubcoreMesh(core_axis_name="c", subcore_axis_name="s")
    fn = pl.kernel(
        functools.partial(_gather_body, nchunk),
        out_type=(
            jax.ShapeDtypeStruct((NW * nchunk, CHUNK, dp), jnp.float32),
            jax.ShapeDtypeStruct((NW * nchunk, CHUNK, dp), jnp.float32),
        ),
        mesh=mesh,
        scratch_types=[
            pltpu.VMEM((nchunk, CHUNK), jnp.int32),
            pltpu.VMEM((nchunk, CHUNK), jnp.int32),
            pltpu.VMEM((nchunk, CHUNK, dp), jnp.float32),
            pltpu.VMEM((nchunk, CHUNK, dp), jnp.float32),
            pltpu.SemaphoreType.DMA,
        ],
        compiler_params=pltpu.CompilerParams(use_tc_tiling_on_sc=False),
    )
    return fn(x0, x1, e)


def _pack_body(e0t_ref, e1t_ref, o_ref):
    d = e0t_ref.shape[0]
    o_ref[:, :d] = jnp.transpose(e0t_ref[...])
    o_ref[:, d:] = jnp.transpose(e1t_ref[...])


@jax.jit
def _tc_pack(e0t, e1t):
    d, v = e0t.shape
    bv = 2048
    grid = (pl.cdiv(v, bv),)
    return pl.pallas_call(
        _pack_body,
        grid=grid,
        in_specs=[
            pl.BlockSpec((d, bv), lambda i: (0, i)),
            pl.BlockSpec((d, bv), lambda i: (0, i)),
        ],
        out_specs=pl.BlockSpec((bv, 2 * d), lambda i: (i, 0)),
        out_shape=jax.ShapeDtypeStruct((v, 2 * d), jnp.float32),
    )(e0t, e1t)


def _mlp_body(a0_ref, a1_ref, w1a_ref, w1b_ref, b1_ref, w2_ref, b2_ref,
              w3_ref, b3_ref, o_ref):
    f32 = jnp.float32
    d = w1a_ref.shape[0]
    h = (jnp.dot(a0_ref[:, :d], w1a_ref[...], preferred_element_type=f32)
         + jnp.dot(a1_ref[:, d:], w1b_ref[...], preferred_element_type=f32)
         + b1_ref[...])
    h = jnp.maximum(h, 0.0)
    h = jnp.dot(h, w2_ref[...], preferred_element_type=f32) + b2_ref[...]
    h = jnp.maximum(h, 0.0)
    o_ref[...] = jnp.dot(h, w3_ref[...], preferred_element_type=f32) + b3_ref[...]


@jax.jit
def _tc_mlp(e0, e1, w1a, w1b, b1, w2, b2, w3, b3):
    bh, dp = e0.shape
    n_out = w3.shape[1]
    bm = 2048
    grid = (bh // bm,)
    full = lambda shape: pl.BlockSpec(shape, lambda i: (0, 0))
    return pl.pallas_call(
        _mlp_body,
        grid=grid,
        in_specs=[
            pl.BlockSpec((bm, dp), lambda i: (i, 0)),
            pl.BlockSpec((bm, dp), lambda i: (i, 0)),
            full(w1a.shape),
            full(w1b.shape),
            full(b1.shape),
            full(w2.shape),
            full(b2.shape),
            full(w3.shape),
            full(b3.shape),
        ],
        out_specs=pl.BlockSpec((bm, n_out), lambda i: (i, 0)),
        out_shape=jax.ShapeDtypeStruct((bh, n_out), jnp.float32),
    )(e0, e1, w1a, w1b, b1, w2, b2, w3, b3)


def kernel(x, E0, E1, W1, b1, W2, b2, W3, b3):
    b = x.shape[0]
    d = E0.shape[1]
    bh = b // 2
    nchunk = bh // (NW * CHUNK)
    Epk = _tc_pack(jnp.swapaxes(E0, 0, 1), jnp.swapaxes(E1, 0, 1))
    x0 = x[:, 0].reshape(2, NW * nchunk, CHUNK)
    x1 = x[:, 1].reshape(2, NW * nchunk, CHUNK)
    w1a, w1b = W1[:d], W1[d:]
    b1r, b2r, b3r = b1.reshape(1, -1), b2.reshape(1, -1), b3.reshape(1, -1)
    outs = []
    for half in range(2):
        emb0, emb1 = _sc_gather(x0[half], x1[half], Epk, bh)
        outs.append(_tc_mlp(emb0.reshape(bh, 2 * d), emb1.reshape(bh, 2 * d),
                            w1a, w1b, b1r, W2, b2r, W3, b3r))
    return jnp.concatenate(outs, axis=0)


# speedup vs baseline: 1.3763x; 1.1200x over previous
import functools

import jax
import jax.numpy as jnp
from jax import lax
from jax.experimental import pallas as pl
from jax.experimental.pallas import tpu as pltpu
from jax.experimental.pallas import tpu_sc as plsc

NC = 2
NS = 16
NW = NC * NS
CHUNK = 128


def _gather_body(nchunk, x0_hbm, x1_hbm, e_hbm, out0_hbm, out1_hbm,
                 idx0_v, idx1_v, rows0_v, rows1_v, sem):
    wid = lax.axis_index("s") * NC + lax.axis_index("c")
    base = wid * nchunk
    pltpu.sync_copy(x0_hbm.at[pl.ds(base, nchunk)], idx0_v)
    pltpu.sync_copy(x1_hbm.at[pl.ds(base, nchunk)], idx1_v)
    copies = []
    for j in range(nchunk):
        copies.append(pltpu.make_async_copy(
            e_hbm.at[idx0_v.at[j]], rows0_v.at[j], sem))
        copies.append(pltpu.make_async_copy(
            e_hbm.at[idx1_v.at[j]], rows1_v.at[j], sem))
    for c in copies:
        c.start()
    for c in copies:
        c.wait()
    pltpu.sync_copy(rows0_v, out0_hbm.at[pl.ds(base, nchunk)])
    pltpu.sync_copy(rows1_v, out1_hbm.at[pl.ds(base, nchunk)])


@functools.partial(jax.jit, static_argnums=(3,))
def _sc_gather(x0, x1, e, bh):
    nchunk = bh // (NW * CHUNK)
    dp = e.shape[1]
    mesh = plsc.VectorSubcoreMesh(core_axis_name="c", subcore_axis_name="s")
    fn = pl.kernel(
        functools.partial(_gather_body, nchunk),
        out_type=(
            jax.ShapeDtypeStruct((NW * nchunk, CHUNK, dp), jnp.float32),
            jax.ShapeDtypeStruct((NW * nchunk, CHUNK, dp), jnp.float32),
        ),
        mesh=mesh,
        scratch_types=[
            pltpu.VMEM((nchunk, CHUNK), jnp.int32),
            pltpu.VMEM((nchunk, CHUNK), jnp.int32),
            pltpu.VMEM((nchunk, CHUNK, dp), jnp.float32),
            pltpu.VMEM((nchunk, CHUNK, dp), jnp.float32),
            pltpu.SemaphoreType.DMA,
        ],
        compiler_params=pltpu.CompilerParams(use_tc_tiling_on_sc=False),
    )
    return fn(x0, x1, e)


def _pack_body(e0t_ref, e1t_ref, o_ref):
    d = e0t_ref.shape[0]
    o_ref[:, :d] = jnp.transpose(e0t_ref[...])
    o_ref[:, d:] = jnp.transpose(e1t_ref[...])


@jax.jit
def _tc_pack(e0t, e1t):
    d, v = e0t.shape
    bv = 4096
    grid = (pl.cdiv(v, bv),)
    return pl.pallas_call(
        _pack_body,
        grid=grid,
        in_specs=[
            pl.BlockSpec((d, bv), lambda i: (0, i)),
            pl.BlockSpec((d, bv), lambda i: (0, i)),
        ],
        out_specs=pl.BlockSpec((bv, 2 * d), lambda i: (i, 0)),
        out_shape=jax.ShapeDtypeStruct((v, 2 * d), jnp.float32),
    )(e0t, e1t)


def _mlp_body(a0_ref, a1_ref, w1a_ref, w1b_ref, b1_ref, w2_ref, b2_ref,
              w3_ref, b3_ref, o_ref):
    f32 = jnp.float32
    d = w1a_ref.shape[0]
    h = (jnp.dot(a0_ref[:, :d], w1a_ref[...], preferred_element_type=f32)
         + jnp.dot(a1_ref[:, d:], w1b_ref[...], preferred_element_type=f32)
         + b1_ref[...])
    h = jnp.maximum(h, 0.0)
    h = jnp.dot(h, w2_ref[...], preferred_element_type=f32) + b2_ref[...]
    h = jnp.maximum(h, 0.0)
    o_ref[...] = jnp.dot(h, w3_ref[...], preferred_element_type=f32) + b3_ref[...]


@jax.jit
def _tc_mlp(e0, e1, w1a, w1b, b1, w2, b2, w3, b3):
    bh, dp = e0.shape
    n_out = w3.shape[1]
    bm = 2048
    grid = (bh // bm,)
    full = lambda shape: pl.BlockSpec(shape, lambda i: (0, 0))
    return pl.pallas_call(
        _mlp_body,
        grid=grid,
        in_specs=[
            pl.BlockSpec((bm, dp), lambda i: (i, 0)),
            pl.BlockSpec((bm, dp), lambda i: (i, 0)),
            full(w1a.shape),
            full(w1b.shape),
            full(b1.shape),
            full(w2.shape),
            full(b2.shape),
            full(w3.shape),
            full(b3.shape),
        ],
        out_specs=pl.BlockSpec((bm, n_out), lambda i: (i, 0)),
        out_shape=jax.ShapeDtypeStruct((bh, n_out), jnp.float32),
    )(e0, e1, w1a, w1b, b1, w2, b2, w3, b3)


def kernel(x, E0, E1, W1, b1, W2, b2, W3, b3):
    b = x.shape[0]
    d = E0.shape[1]
    bh = b // 2
    nchunk = bh // (NW * CHUNK)
    Epk = _tc_pack(jnp.swapaxes(E0, 0, 1), jnp.swapaxes(E1, 0, 1))
    x0 = x[:, 0].reshape(2, NW * nchunk, CHUNK)
    x1 = x[:, 1].reshape(2, NW * nchunk, CHUNK)
    w1a, w1b = W1[:d], W1[d:]
    b1r, b2r, b3r = b1.reshape(1, -1), b2.reshape(1, -1), b3.reshape(1, -1)
    outs = []
    for half in range(2):
        emb0, emb1 = _sc_gather(x0[half], x1[half], Epk, bh)
        outs.append(_tc_mlp(emb0.reshape(bh, 2 * d), emb1.reshape(bh, 2 * d),
                            w1a, w1b, b1r, W2, b2r, W3, b3r))
    return jnp.concatenate(outs, axis=0)


# speedup vs baseline: 1.4368x; 1.0439x over previous
import functools

import jax
import jax.numpy as jnp
from jax import lax
from jax.experimental import pallas as pl
from jax.experimental.pallas import tpu as pltpu
from jax.experimental.pallas import tpu_sc as plsc

NC = 2
NS = 16
NW = NC * NS
CHUNK = 128


def _gather_body(nchunk, x0_hbm, x1_hbm, e_hbm, out0_hbm, out1_hbm,
                 idx0_v, idx1_v, rows0_v, rows1_v, sem):
    wid = lax.axis_index("s") * NC + lax.axis_index("c")
    base = wid * nchunk
    pltpu.sync_copy(x0_hbm.at[pl.ds(base, nchunk)], idx0_v)
    pltpu.sync_copy(x1_hbm.at[pl.ds(base, nchunk)], idx1_v)
    copies = []
    for j in range(nchunk):
        copies.append(pltpu.make_async_copy(
            e_hbm.at[idx0_v.at[j]], rows0_v.at[j], sem))
        copies.append(pltpu.make_async_copy(
            e_hbm.at[idx1_v.at[j]], rows1_v.at[j], sem))
    for c in copies:
        c.start()
    for c in copies:
        c.wait()
    pltpu.sync_copy(rows0_v, out0_hbm.at[pl.ds(base, nchunk)])
    pltpu.sync_copy(rows1_v, out1_hbm.at[pl.ds(base, nchunk)])


@functools.partial(jax.jit, static_argnums=(3,))
def _sc_gather(x0, x1, e, bh):
    nchunk = bh // (NW * CHUNK)
    dp = e.shape[1]
    mesh = plsc.VectorSubcoreMesh(core_axis_name="c", subcore_axis_name="s")
    fn = pl.kernel(
        functools.partial(_gather_body, nchunk),
        out_type=(
            jax.ShapeDtypeStruct((NW * nchunk, CHUNK, dp), jnp.float32),
            jax.ShapeDtypeStruct((NW * nchunk, CHUNK, dp), jnp.float32),
        ),
        mesh=mesh,
        scratch_types=[
            pltpu.VMEM((nchunk, CHUNK), jnp.int32),
            pltpu.VMEM((nchunk, CHUNK), jnp.int32),
            pltpu.VMEM((nchunk, CHUNK, dp), jnp.float32),
            pltpu.VMEM((nchunk, CHUNK, dp), jnp.float32),
            pltpu.SemaphoreType.DMA,
        ],
        compiler_params=pltpu.CompilerParams(use_tc_tiling_on_sc=False),
    )
    return fn(x0, x1, e)


def _pack_body(e0t_ref, e1t_ref, o_ref):
    d = e0t_ref.shape[0]
    o_ref[:, :d] = jnp.transpose(e0t_ref[...])
    o_ref[:, d:] = jnp.transpose(e1t_ref[...])


@jax.jit
def _tc_pack(e0t, e1t):
    d, v = e0t.shape
    bv = 8192
    grid = (pl.cdiv(v, bv),)
    return pl.pallas_call(
        _pack_body,
        grid=grid,
        in_specs=[
            pl.BlockSpec((d, bv), lambda i: (0, i)),
            pl.BlockSpec((d, bv), lambda i: (0, i)),
        ],
        out_specs=pl.BlockSpec((bv, 2 * d), lambda i: (i, 0)),
        out_shape=jax.ShapeDtypeStruct((v, 2 * d), jnp.float32),
    )(e0t, e1t)


def _mlp_body(a0_ref, a1_ref, w1a_ref, w1b_ref, b1_ref, w2_ref, b2_ref,
              w3_ref, b3_ref, o_ref):
    f32 = jnp.float32
    d = w1a_ref.shape[0]
    h = (jnp.dot(a0_ref[:, :d], w1a_ref[...], preferred_element_type=f32)
         + jnp.dot(a1_ref[:, d:], w1b_ref[...], preferred_element_type=f32)
         + b1_ref[...])
    h = jnp.maximum(h, 0.0)
    h = jnp.dot(h, w2_ref[...], preferred_element_type=f32) + b2_ref[...]
    h = jnp.maximum(h, 0.0)
    o_ref[...] = jnp.dot(h, w3_ref[...], preferred_element_type=f32) + b3_ref[...]


@jax.jit
def _tc_mlp(e0, e1, w1a, w1b, b1, w2, b2, w3, b3):
    bh, dp = e0.shape
    n_out = w3.shape[1]
    bm = 2048
    grid = (bh // bm,)
    full = lambda shape: pl.BlockSpec(shape, lambda i: (0, 0))
    return pl.pallas_call(
        _mlp_body,
        grid=grid,
        in_specs=[
            pl.BlockSpec((bm, dp), lambda i: (i, 0)),
            pl.BlockSpec((bm, dp), lambda i: (i, 0)),
            full(w1a.shape),
            full(w1b.shape),
            full(b1.shape),
            full(w2.shape),
            full(b2.shape),
            full(w3.shape),
            full(b3.shape),
        ],
        out_specs=pl.BlockSpec((bm, n_out), lambda i: (i, 0)),
        out_shape=jax.ShapeDtypeStruct((bh, n_out), jnp.float32),
    )(e0, e1, w1a, w1b, b1, w2, b2, w3, b3)


def kernel(x, E0, E1, W1, b1, W2, b2, W3, b3):
    b = x.shape[0]
    d = E0.shape[1]
    bh = b // 2
    nchunk = bh // (NW * CHUNK)
    Epk = _tc_pack(jnp.swapaxes(E0, 0, 1), jnp.swapaxes(E1, 0, 1))
    x0 = x[:, 0].reshape(2, NW * nchunk, CHUNK)
    x1 = x[:, 1].reshape(2, NW * nchunk, CHUNK)
    w1a, w1b = W1[:d], W1[d:]
    b1r, b2r, b3r = b1.reshape(1, -1), b2.reshape(1, -1), b3.reshape(1, -1)
    outs = []
    for half in range(2):
        emb0, emb1 = _sc_gather(x0[half], x1[half], Epk, bh)
        outs.append(_tc_mlp(emb0.reshape(bh, 2 * d), emb1.reshape(bh, 2 * d),
                            w1a, w1b, b1r, W2, b2r, W3, b3r))
    return jnp.concatenate(outs, axis=0)
